# Initial kernel scaffold; baseline (speedup 1.0000x reference)
import jax
import jax.numpy as jnp
from jax.experimental import pallas as pl


def kernel(x, edge_index, edge_attr, p1_W1, p1_b1, p1_W2, p1_b2, p2_W1, p2_b1, p2_W2, p2_b2, cls_W, cls_b):
    raise NotImplementedError("write your pallas kernel here")


# speedup vs baseline: 4.4025x; 4.4025x over previous
import functools

import jax
import jax.numpy as jnp
from jax import lax
from jax.experimental import pallas as pl
from jax.experimental.pallas import tpu as pltpu
from jax.experimental.pallas import tpu_sc as plsc

N_NODES = 100000
N_EDGES = 6400000
F = 16
NS = 16
K = 128
SUB = 5
EPT = N_EDGES // NS
BLK_PER_TILE = EPT // K
OUTER = BLK_PER_TILE // SUB
ROWS_PER_TILE = N_NODES // NS
ZCH = 250


def _edge_kernel_call(src2d, dst1d, ea2d, tables, coef):
    mesh = plsc.VectorSubcoreMesh(core_axis_name="c", subcore_axis_name="s")

    @functools.partial(
        pl.kernel,
        mesh=mesh,
        compiler_params=pltpu.CompilerParams(use_tc_tiling_on_sc=False),
        out_type=jax.ShapeDtypeStruct((2 * N_NODES, F), jnp.float32),
        scratch_types=[
            pltpu.VMEM((SUB, K), jnp.int32),
            pltpu.VMEM((K,), jnp.int32),
            pltpu.VMEM((SUB, 3 * K), jnp.float32),
            pltpu.VMEM((K, F), jnp.float32),
            pltpu.VMEM((K, F), jnp.float32),
            pltpu.VMEM((ZCH, F), jnp.float32),
            pltpu.VMEM((3, F), jnp.float32),
            pltpu.VMEM_SHARED((N_NODES, F), jnp.float32),
            pltpu.SemaphoreType.DMA,
        ],
    )
    def k(src_hbm, dst_hbm, ea_hbm, tab_hbm, coef_hbm, out_hbm,
          src_v, dst_v, ea_v, rows_v, z_v, zbuf_v, coef_v, acc_sh, sem):
        c = lax.axis_index("c")
        s = lax.axis_index("s")

        def zrow(i, _):
            zbuf_v[i, :] = jnp.zeros((F,), jnp.float32)
            return _
        lax.fori_loop(0, ZCH, zrow, 0)

        def zcopy(t, _):
            pltpu.sync_copy(zbuf_v, acc_sh.at[pl.ds(s * ROWS_PER_TILE + t * ZCH, ZCH)])
            return _
        lax.fori_loop(0, ROWS_PER_TILE // ZCH, zcopy, 0)

        pltpu.sync_copy(coef_hbm.at[c], coef_v)
        plsc.subcore_barrier()

        c0 = coef_v[0, :]
        c1 = coef_v[1, :]
        c2 = coef_v[2, :]
        row_off = c * N_NODES

        def outer(i, _):
            blk0 = s * BLK_PER_TILE + i * SUB
            pltpu.sync_copy(src_hbm.at[pl.ds(blk0, SUB)], src_v)
            pltpu.sync_copy(ea_hbm.at[pl.ds(blk0, SUB)], ea_v)

            def shift(t, _):
                jj = t // (K // 16)
                ll = (t % (K // 16)) * 16
                src_v[jj, pl.ds(ll, 16)] = src_v[jj, pl.ds(ll, 16)] + row_off
                return _
            lax.fori_loop(0, SUB * (K // 16), shift, 0)

            for j in range(SUB):
                pltpu.sync_copy(dst_hbm.at[pl.ds((blk0 + j) * K, K)], dst_v)
                pltpu.async_copy(tab_hbm.at[src_v.at[j]], rows_v, sem).wait()

                def edge(kk, _):
                    base = 24 * kk
                    va = ea_v[j, pl.ds(base, 16)]
                    vb = ea_v[j, pl.ds(base + 8, 16)]

                    def attr(t):
                        return va[t] if t < 16 else vb[t - 8]

                    for u in range(8):
                        e = kk * 8 + u
                        g = rows_v[e, :]
                        e0 = attr(3 * u)
                        e1 = attr(3 * u + 1)
                        e2 = attr(3 * u + 2)
                        z_v[e, :] = jnp.maximum(g + e0 * c0 + e1 * c1 + e2 * c2, 0.0)
                    return _
                lax.fori_loop(0, K // 8, edge, 0)

                pltpu.sync_copy(z_v, acc_sh.at[dst_v], add=True)
            return _
        lax.fori_loop(0, OUTER, outer, 0)

        plsc.subcore_barrier()
        r0 = s * ROWS_PER_TILE
        pltpu.sync_copy(acc_sh.at[pl.ds(r0, ROWS_PER_TILE)],
                        out_hbm.at[pl.ds(row_off + r0, ROWS_PER_TILE)])

    return k(src2d, dst1d, ea2d, tables, coef)


BN = 10000


def _hi_chunk(pre):
    n = pre.shape[0]
    return jnp.concatenate(
        [pre[:, 16:20],
         jnp.ones((n, 1), jnp.float32),
         jnp.zeros((n, 11), jnp.float32)], axis=1)


def _prep1_body(x_ref, w_ref, b_ref, o_ref):
    xb = x_ref[...]
    w = w_ref[...]
    pre = lax.dot_general(xb, w[:, :4], (((1,), (1,)), ((), ())),
                          preferred_element_type=jnp.float32) + b_ref[...][None, :]
    o_ref[0] = pre[:, :16]
    o_ref[1] = _hi_chunk(pre)


def _mean_head(acc_ref, w2_ref, b2_ref):
    lo = acc_ref[0]
    hi = acc_ref[1]
    cnt = hi[:, 4]
    ssum = jnp.concatenate([lo, hi[:, :4]], axis=1)
    mean = ssum / jnp.maximum(cnt, 1.0)[:, None]
    h = lax.dot_general(mean, w2_ref[...], (((1,), (1,)), ((), ())),
                        preferred_element_type=jnp.float32)
    return h + b2_ref[...][None, :] * jnp.minimum(cnt, 1.0)[:, None]


def _mid_body(acc_ref, w2_ref, b2_ref, w1n_ref, b1n_ref, o_ref):
    h = jnp.maximum(_mean_head(acc_ref, w2_ref, b2_ref), 0.0)
    w1n = w1n_ref[...]
    pre = lax.dot_general(h, w1n[:, :20], (((1,), (1,)), ((), ())),
                          preferred_element_type=jnp.float32) + b1n_ref[...][None, :]
    o_ref[0] = pre[:, :16]
    o_ref[1] = _hi_chunk(pre)


def _final_body(acc_ref, w2_ref, b2_ref, cw_ref, cb_ref, o_ref):
    h = _mean_head(acc_ref, w2_ref, b2_ref)
    logits = lax.dot_general(h, cw_ref[...], (((1,), (1,)), ((), ())),
                             preferred_element_type=jnp.float32) + cb_ref[...][None, :]
    o_ref[...] = jax.nn.sigmoid(logits)


def _full(shape):
    return pl.BlockSpec(shape, lambda i: tuple(0 for _ in shape))


def _node_call(body, ins, in_specs, out_shape, out_spec):
    return pl.pallas_call(
        body,
        grid=(N_NODES // BN,),
        in_specs=in_specs,
        out_specs=out_spec,
        out_shape=out_shape,
    )(*ins)


def kernel(x, edge_index, edge_attr, p1_W1, p1_b1, p1_W2, p1_b2,
           p2_W1, p2_b1, p2_W2, p2_b2, cls_W, cls_b):
    src = edge_index[0]
    dst = edge_index[1]
    src2d = src.reshape(N_EDGES // K, K)
    ea2d = edge_attr.reshape(N_EDGES // K, 3 * K)

    def coef_of(W1, dh):
        C = W1[:, dh:]
        lo = C[:16].T
        hi = jnp.pad(C[16:20].T, ((0, 0), (0, 12)))
        return jnp.stack([lo, hi])

    tab_spec = pl.BlockSpec((2, BN, F), lambda i: (0, i, 0))
    acc_spec = pl.BlockSpec((2, BN, F), lambda i: (0, i, 0))

    tables1 = _node_call(
        _prep1_body,
        (x, p1_W1, p1_b1),
        [pl.BlockSpec((BN, 4), lambda i: (i, 0)), _full(p1_W1.shape), _full(p1_b1.shape)],
        jax.ShapeDtypeStruct((2, N_NODES, F), jnp.float32),
        tab_spec,
    )
    acc1 = _edge_kernel_call(src2d, dst, ea2d,
                             tables1.reshape(2 * N_NODES, F), coef_of(p1_W1, 4))
    acc1 = acc1.reshape(2, N_NODES, F)

    tables2 = _node_call(
        _mid_body,
        (acc1, p1_W2, p1_b2, p2_W1, p2_b1),
        [acc_spec, _full(p1_W2.shape), _full(p1_b2.shape),
         _full(p2_W1.shape), _full(p2_b1.shape)],
        jax.ShapeDtypeStruct((2, N_NODES, F), jnp.float32),
        tab_spec,
    )
    acc2 = _edge_kernel_call(src2d, dst, ea2d,
                             tables2.reshape(2 * N_NODES, F), coef_of(p2_W1, 20))
    acc2 = acc2.reshape(2, N_NODES, F)

    return _node_call(
        _final_body,
        (acc2, p2_W2, p2_b2, cls_W, cls_b),
        [acc_spec, _full(p2_W2.shape), _full(p2_b2.shape),
         _full(cls_W.shape), _full(cls_b.shape)],
        jax.ShapeDtypeStruct((N_NODES, 3), jnp.float32),
        pl.BlockSpec((BN, 3), lambda i: (i, 0)),
    )


# speedup vs baseline: 9.8982x; 2.2483x over previous
import functools

import jax
import jax.numpy as jnp
from jax import lax
from jax.experimental import pallas as pl
from jax.experimental.pallas import tpu as pltpu
from jax.experimental.pallas import tpu_sc as plsc

N_NODES = 100000
N_EDGES = 6400000
F = 16
NS = 16
K = 128
SUB = 25
CH = SUB * K
NBLK = N_EDGES // (NS * CH)
BLK_PER_TILE = N_EDGES // (NS * K)
ROWS_PER_TILE = N_NODES // NS
ZCH = 250


def _edge_kernel_call(src2d, dst2d, ea_t, tables, coef):
    mesh = plsc.VectorSubcoreMesh(core_axis_name="c", subcore_axis_name="s")

    @functools.partial(
        pl.kernel,
        mesh=mesh,
        compiler_params=pltpu.CompilerParams(use_tc_tiling_on_sc=False),
        out_type=jax.ShapeDtypeStruct((2 * N_NODES, F), jnp.float32),
        scratch_types=[
            pltpu.VMEM((SUB, K), jnp.int32),
            pltpu.VMEM((SUB, K), jnp.int32),
            [pltpu.VMEM((CH,), jnp.float32)] * 3,
            [pltpu.VMEM((K, F), jnp.float32)] * 2,
            [pltpu.VMEM((K, F), jnp.float32)] * 2,
            pltpu.VMEM((ZCH, F), jnp.float32),
            pltpu.VMEM((3, F), jnp.float32),
            pltpu.VMEM_SHARED((N_NODES, F), jnp.float32),
            [pltpu.SemaphoreType.DMA] * 2,
            [pltpu.SemaphoreType.DMA] * 2,
        ],
    )
    def k(src_hbm, dst_hbm, ea_hbm, tab_hbm, coef_hbm, out_hbm,
          src_v, dst_v, ea_v, rows_v, z_v, zbuf_v, coef_v, acc_sh, sem_g, sem_s):
        c = lax.axis_index("c")
        s = lax.axis_index("s")

        def zrow(i, _):
            zbuf_v[i, :] = jnp.zeros((F,), jnp.float32)
            return _
        lax.fori_loop(0, ZCH, zrow, 0)

        def zcopy(t, _):
            pltpu.sync_copy(zbuf_v, acc_sh.at[pl.ds(s * ROWS_PER_TILE + t * ZCH, ZCH)])
            return _
        lax.fori_loop(0, ROWS_PER_TILE // ZCH, zcopy, 0)

        pltpu.sync_copy(coef_hbm.at[c], coef_v)
        plsc.subcore_barrier()

        c0 = coef_v[0, :]
        c1 = coef_v[1, :]
        c2 = coef_v[2, :]
        row_off = c * N_NODES

        def issue_gather(j, b):
            pltpu.async_copy(tab_hbm.at[src_v.at[j]], rows_v[b], sem_g[b])

        def wait_gather(j, b):
            pltpu.make_async_copy(tab_hbm.at[src_v.at[j]], rows_v[b], sem_g[b]).wait()

        def issue_scatter(j, b):
            pltpu.async_copy(z_v[b], acc_sh.at[dst_v.at[j]], sem_s[b], add=True)

        def wait_scatter(j, b):
            pltpu.make_async_copy(z_v[b], acc_sh.at[dst_v.at[j]], sem_s[b]).wait()

        def compute(j, b):
            def grp(g, _):
                off = j * K + 16 * g
                ve0 = ea_v[0][pl.ds(off, 16)]
                ve1 = ea_v[1][pl.ds(off, 16)]
                ve2 = ea_v[2][pl.ds(off, 16)]
                for u in range(16):
                    e = 16 * g + u
                    gv = rows_v[b][e, :]
                    z_v[b][e, :] = jnp.maximum(
                        gv + ve0[u] * c0 + ve1[u] * c1 + ve2[u] * c2, 0.0)
                return _
            lax.fori_loop(0, K // 16, grp, 0)

        def block(i, _):
            blk0 = s * BLK_PER_TILE + i * SUB
            pltpu.sync_copy(src_hbm.at[pl.ds(blk0, SUB)], src_v)
            pltpu.sync_copy(dst_hbm.at[pl.ds(blk0, SUB)], dst_v)
            for t in range(3):
                pltpu.sync_copy(ea_hbm.at[t, pl.ds(blk0 * K, CH)], ea_v[t])

            def shift(t, _):
                r = t // (K // 16)
                l = (t % (K // 16)) * 16
                src_v[r, pl.ds(l, 16)] = src_v[r, pl.ds(l, 16)] + row_off
                return _
            lax.fori_loop(0, SUB * (K // 16), shift, 0)

            issue_gather(0, 0)

            def pair(jj, _):
                for b in range(2):
                    j = 2 * jj + b
                    wait_gather(j, b)
                    issue_gather(j + 1, 1 - b)

                    @pl.when(jj > 0)
                    def _w():
                        wait_scatter(j - 2, b)
                    compute(j, b)
                    issue_scatter(j, b)
                return _
            lax.fori_loop(0, (SUB - 1) // 2, pair, 0)

            wait_gather(SUB - 1, 0)
            wait_scatter(SUB - 3, 0)
            compute(SUB - 1, 0)
            issue_scatter(SUB - 1, 0)
            wait_scatter(SUB - 1, 0)
            wait_scatter(SUB - 2, 1)
            return _
        lax.fori_loop(0, NBLK, block, 0)

        plsc.subcore_barrier()
        r0 = s * ROWS_PER_TILE
        pltpu.sync_copy(acc_sh.at[pl.ds(r0, ROWS_PER_TILE)],
                        out_hbm.at[pl.ds(row_off + r0, ROWS_PER_TILE)])

    return k(src2d, dst2d, ea_t, tables, coef)


BN = 10000


def _hi_chunk(pre):
    n = pre.shape[0]
    return jnp.concatenate(
        [pre[:, 16:20],
         jnp.ones((n, 1), jnp.float32),
         jnp.zeros((n, 11), jnp.float32)], axis=1)


def _prep1_body(x_ref, w_ref, b_ref, o_ref):
    xb = x_ref[...]
    w = w_ref[...]
    pre = lax.dot_general(xb, w[:, :4], (((1,), (1,)), ((), ())),
                          preferred_element_type=jnp.float32) + b_ref[...][None, :]
    o_ref[0] = pre[:, :16]
    o_ref[1] = _hi_chunk(pre)


def _mean_head(acc_ref, w2_ref, b2_ref):
    lo = acc_ref[0]
    hi = acc_ref[1]
    cnt = hi[:, 4]
    ssum = jnp.concatenate([lo, hi[:, :4]], axis=1)
    mean = ssum / jnp.maximum(cnt, 1.0)[:, None]
    h = lax.dot_general(mean, w2_ref[...], (((1,), (1,)), ((), ())),
                        preferred_element_type=jnp.float32)
    return h + b2_ref[...][None, :] * jnp.minimum(cnt, 1.0)[:, None]


def _mid_body(acc_ref, w2_ref, b2_ref, w1n_ref, b1n_ref, o_ref):
    h = jnp.maximum(_mean_head(acc_ref, w2_ref, b2_ref), 0.0)
    w1n = w1n_ref[...]
    pre = lax.dot_general(h, w1n[:, :20], (((1,), (1,)), ((), ())),
                          preferred_element_type=jnp.float32) + b1n_ref[...][None, :]
    o_ref[0] = pre[:, :16]
    o_ref[1] = _hi_chunk(pre)


def _final_body(acc_ref, w2_ref, b2_ref, cw_ref, cb_ref, o_ref):
    h = _mean_head(acc_ref, w2_ref, b2_ref)
    logits = lax.dot_general(h, cw_ref[...], (((1,), (1,)), ((), ())),
                             preferred_element_type=jnp.float32) + cb_ref[...][None, :]
    o_ref[...] = jax.nn.sigmoid(logits)


def _full(shape):
    return pl.BlockSpec(shape, lambda i: tuple(0 for _ in shape))


def _node_call(body, ins, in_specs, out_shape, out_spec):
    return pl.pallas_call(
        body,
        grid=(N_NODES // BN,),
        in_specs=in_specs,
        out_specs=out_spec,
        out_shape=out_shape,
    )(*ins)


def kernel(x, edge_index, edge_attr, p1_W1, p1_b1, p1_W2, p1_b2,
           p2_W1, p2_b1, p2_W2, p2_b2, cls_W, cls_b):
    src = edge_index[0]
    dst = edge_index[1]
    src2d = src.reshape(N_EDGES // K, K)
    dst2d = dst.reshape(N_EDGES // K, K)
    ea_t = edge_attr.T

    def coef_of(W1, dh):
        C = W1[:, dh:]
        lo = C[:16].T
        hi = jnp.pad(C[16:20].T, ((0, 0), (0, 12)))
        return jnp.stack([lo, hi])

    tab_spec = pl.BlockSpec((2, BN, F), lambda i: (0, i, 0))
    acc_spec = pl.BlockSpec((2, BN, F), lambda i: (0, i, 0))

    tables1 = _node_call(
        _prep1_body,
        (x, p1_W1, p1_b1),
        [pl.BlockSpec((BN, 4), lambda i: (i, 0)), _full(p1_W1.shape), _full(p1_b1.shape)],
        jax.ShapeDtypeStruct((2, N_NODES, F), jnp.float32),
        tab_spec,
    )
    acc1 = _edge_kernel_call(src2d, dst2d, ea_t,
                             tables1.reshape(2 * N_NODES, F), coef_of(p1_W1, 4))
    acc1 = acc1.reshape(2, N_NODES, F)

    tables2 = _node_call(
        _mid_body,
        (acc1, p1_W2, p1_b2, p2_W1, p2_b1),
        [acc_spec, _full(p1_W2.shape), _full(p1_b2.shape),
         _full(p2_W1.shape), _full(p2_b1.shape)],
        jax.ShapeDtypeStruct((2, N_NODES, F), jnp.float32),
        tab_spec,
    )
    acc2 = _edge_kernel_call(src2d, dst2d, ea_t,
                             tables2.reshape(2 * N_NODES, F), coef_of(p2_W1, 20))
    acc2 = acc2.reshape(2, N_NODES, F)

    return _node_call(
        _final_body,
        (acc2, p2_W2, p2_b2, cls_W, cls_b),
        [acc_spec, _full(p2_W2.shape), _full(p2_b2.shape),
         _full(cls_W.shape), _full(cls_b.shape)],
        jax.ShapeDtypeStruct((N_NODES, 3), jnp.float32),
        pl.BlockSpec((BN, 3), lambda i: (i, 0)),
    )


# speedup vs baseline: 12.1378x; 1.2263x over previous
import functools

import jax
import jax.numpy as jnp
from jax import lax
from jax.experimental import pallas as pl
from jax.experimental.pallas import tpu as pltpu
from jax.experimental.pallas import tpu_sc as plsc

N_NODES = 100000
N_EDGES = 6400000
F = 16
NS = 16
K = 128
SUB = 25
CH = SUB * K
NBLK = N_EDGES // (NS * CH)
BLK_PER_TILE = N_EDGES // (NS * K)
ROWS_PER_TILE = N_NODES // NS
ZCH = 250


def _edge_kernel_call(src2d, dst2d, ea_t, tables, coef):
    mesh = plsc.VectorSubcoreMesh(core_axis_name="c", subcore_axis_name="s")

    @functools.partial(
        pl.kernel,
        mesh=mesh,
        compiler_params=pltpu.CompilerParams(use_tc_tiling_on_sc=False),
        out_type=jax.ShapeDtypeStruct((2 * N_NODES, F), jnp.float32),
        scratch_types=[
            pltpu.VMEM((SUB, K), jnp.int32),
            pltpu.VMEM((SUB, K), jnp.int32),
            [pltpu.VMEM((CH,), jnp.float32)] * 3,
            [pltpu.VMEM((K, F), jnp.float32)] * 2,
            [pltpu.VMEM((K, F), jnp.float32)] * 2,
            pltpu.VMEM((ZCH, F), jnp.float32),
            pltpu.VMEM((3, F), jnp.float32),
            pltpu.VMEM_SHARED((N_NODES, F), jnp.float32),
            [pltpu.SemaphoreType.DMA] * 2,
            [pltpu.SemaphoreType.DMA] * 2,
        ],
    )
    def k(src_hbm, dst_hbm, ea_hbm, tab_hbm, coef_hbm, out_hbm,
          src_v, dst_v, ea_v, rows_v, z_v, zbuf_v, coef_v, acc_sh, sem_g, sem_s):
        c = lax.axis_index("c")
        s = lax.axis_index("s")

        @plsc.parallel_loop(0, ZCH, 1, unroll=4)
        def zrow(i):
            zbuf_v[i, :] = jnp.zeros((F,), jnp.float32)

        def zcopy(t, _):
            pltpu.sync_copy(zbuf_v, acc_sh.at[pl.ds(s * ROWS_PER_TILE + t * ZCH, ZCH)])
            return _
        lax.fori_loop(0, ROWS_PER_TILE // ZCH, zcopy, 0)

        pltpu.sync_copy(coef_hbm.at[c], coef_v)
        plsc.subcore_barrier()

        c0 = coef_v[0, :]
        c1 = coef_v[1, :]
        c2 = coef_v[2, :]
        row_off = c * N_NODES

        def issue_gather(j, b):
            pltpu.async_copy(tab_hbm.at[src_v.at[j]], rows_v[b], sem_g[b])

        def wait_gather(j, b):
            pltpu.make_async_copy(tab_hbm.at[src_v.at[j]], rows_v[b], sem_g[b]).wait()

        def issue_scatter(j, b):
            pltpu.async_copy(z_v[b], acc_sh.at[dst_v.at[j]], sem_s[b], add=True)

        def wait_scatter(j, b):
            pltpu.make_async_copy(z_v[b], acc_sh.at[dst_v.at[j]], sem_s[b]).wait()

        def compute(j, b):
            @plsc.parallel_loop(0, K // 16, 1, unroll=2)
            def grp(g):
                off = j * K + 16 * g
                ve0 = ea_v[0][pl.ds(off, 16)]
                ve1 = ea_v[1][pl.ds(off, 16)]
                ve2 = ea_v[2][pl.ds(off, 16)]
                for u in range(16):
                    e = 16 * g + u
                    gv = rows_v[b][e, :]
                    z_v[b][e, :] = jnp.maximum(
                        gv + ve0[u] * c0 + ve1[u] * c1 + ve2[u] * c2, 0.0)

        def block(i, _):
            blk0 = s * BLK_PER_TILE + i * SUB
            pltpu.sync_copy(src_hbm.at[pl.ds(blk0, SUB)], src_v)
            pltpu.sync_copy(dst_hbm.at[pl.ds(blk0, SUB)], dst_v)
            for t in range(3):
                pltpu.sync_copy(ea_hbm.at[t, pl.ds(blk0 * K, CH)], ea_v[t])

            @plsc.parallel_loop(0, SUB * (K // 16), 1, unroll=4)
            def shift(t):
                r = t // (K // 16)
                l = (t % (K // 16)) * 16
                src_v[r, pl.ds(l, 16)] = src_v[r, pl.ds(l, 16)] + row_off

            issue_gather(0, 0)

            def pair(jj, _):
                for b in range(2):
                    j = 2 * jj + b
                    wait_gather(j, b)
                    issue_gather(j + 1, 1 - b)

                    @pl.when(jj > 0)
                    def _w():
                        wait_scatter(j - 2, b)
                    compute(j, b)
                    issue_scatter(j, b)
                return _
            lax.fori_loop(0, (SUB - 1) // 2, pair, 0)

            wait_gather(SUB - 1, 0)
            wait_scatter(SUB - 3, 0)
            compute(SUB - 1, 0)
            issue_scatter(SUB - 1, 0)
            wait_scatter(SUB - 1, 0)
            wait_scatter(SUB - 2, 1)
            return _
        lax.fori_loop(0, NBLK, block, 0)

        plsc.subcore_barrier()
        r0 = s * ROWS_PER_TILE
        pltpu.sync_copy(acc_sh.at[pl.ds(r0, ROWS_PER_TILE)],
                        out_hbm.at[pl.ds(row_off + r0, ROWS_PER_TILE)])

    return k(src2d, dst2d, ea_t, tables, coef)


BN = 10000


def _hi_chunk(pre):
    n = pre.shape[0]
    return jnp.concatenate(
        [pre[:, 16:20],
         jnp.ones((n, 1), jnp.float32),
         jnp.zeros((n, 11), jnp.float32)], axis=1)


def _prep1_body(x_ref, w_ref, b_ref, o_ref):
    xb = x_ref[...]
    w = w_ref[...]
    pre = lax.dot_general(xb, w[:, :4], (((1,), (1,)), ((), ())),
                          preferred_element_type=jnp.float32) + b_ref[...][None, :]
    o_ref[0] = pre[:, :16]
    o_ref[1] = _hi_chunk(pre)


def _mean_head(acc_ref, w2_ref, b2_ref):
    lo = acc_ref[0]
    hi = acc_ref[1]
    cnt = hi[:, 4]
    ssum = jnp.concatenate([lo, hi[:, :4]], axis=1)
    mean = ssum / jnp.maximum(cnt, 1.0)[:, None]
    h = lax.dot_general(mean, w2_ref[...], (((1,), (1,)), ((), ())),
                        preferred_element_type=jnp.float32)
    return h + b2_ref[...][None, :] * jnp.minimum(cnt, 1.0)[:, None]


def _mid_body(acc_ref, w2_ref, b2_ref, w1n_ref, b1n_ref, o_ref):
    h = jnp.maximum(_mean_head(acc_ref, w2_ref, b2_ref), 0.0)
    w1n = w1n_ref[...]
    pre = lax.dot_general(h, w1n[:, :20], (((1,), (1,)), ((), ())),
                          preferred_element_type=jnp.float32) + b1n_ref[...][None, :]
    o_ref[0] = pre[:, :16]
    o_ref[1] = _hi_chunk(pre)


def _final_body(acc_ref, w2_ref, b2_ref, cw_ref, cb_ref, o_ref):
    h = _mean_head(acc_ref, w2_ref, b2_ref)
    logits = lax.dot_general(h, cw_ref[...], (((1,), (1,)), ((), ())),
                             preferred_element_type=jnp.float32) + cb_ref[...][None, :]
    o_ref[...] = jax.nn.sigmoid(logits)


def _full(shape):
    return pl.BlockSpec(shape, lambda i: tuple(0 for _ in shape))


def _node_call(body, ins, in_specs, out_shape, out_spec):
    return pl.pallas_call(
        body,
        grid=(N_NODES // BN,),
        in_specs=in_specs,
        out_specs=out_spec,
        out_shape=out_shape,
    )(*ins)


def kernel(x, edge_index, edge_attr, p1_W1, p1_b1, p1_W2, p1_b2,
           p2_W1, p2_b1, p2_W2, p2_b2, cls_W, cls_b):
    src = edge_index[0]
    dst = edge_index[1]
    src2d = src.reshape(N_EDGES // K, K)
    dst2d = dst.reshape(N_EDGES // K, K)
    ea_t = edge_attr.T

    def coef_of(W1, dh):
        C = W1[:, dh:]
        lo = C[:16].T
        hi = jnp.pad(C[16:20].T, ((0, 0), (0, 12)))
        return jnp.stack([lo, hi])

    tab_spec = pl.BlockSpec((2, BN, F), lambda i: (0, i, 0))
    acc_spec = pl.BlockSpec((2, BN, F), lambda i: (0, i, 0))

    tables1 = _node_call(
        _prep1_body,
        (x, p1_W1, p1_b1),
        [pl.BlockSpec((BN, 4), lambda i: (i, 0)), _full(p1_W1.shape), _full(p1_b1.shape)],
        jax.ShapeDtypeStruct((2, N_NODES, F), jnp.float32),
        tab_spec,
    )
    acc1 = _edge_kernel_call(src2d, dst2d, ea_t,
                             tables1.reshape(2 * N_NODES, F), coef_of(p1_W1, 4))
    acc1 = acc1.reshape(2, N_NODES, F)

    tables2 = _node_call(
        _mid_body,
        (acc1, p1_W2, p1_b2, p2_W1, p2_b1),
        [acc_spec, _full(p1_W2.shape), _full(p1_b2.shape),
         _full(p2_W1.shape), _full(p2_b1.shape)],
        jax.ShapeDtypeStruct((2, N_NODES, F), jnp.float32),
        tab_spec,
    )
    acc2 = _edge_kernel_call(src2d, dst2d, ea_t,
                             tables2.reshape(2 * N_NODES, F), coef_of(p2_W1, 20))
    acc2 = acc2.reshape(2, N_NODES, F)

    return _node_call(
        _final_body,
        (acc2, p2_W2, p2_b2, cls_W, cls_b),
        [acc_spec, _full(p2_W2.shape), _full(p2_b2.shape),
         _full(cls_W.shape), _full(cls_b.shape)],
        jax.ShapeDtypeStruct((N_NODES, 3), jnp.float32),
        pl.BlockSpec((BN, 3), lambda i: (i, 0)),
    )


# speedup vs baseline: 16.8652x; 1.3895x over previous
import functools

import jax
import jax.numpy as jnp
from jax import lax
from jax.experimental import pallas as pl
from jax.experimental.pallas import tpu as pltpu
from jax.experimental.pallas import tpu_sc as plsc

N_NODES = 100000
N_EDGES = 6400000
F = 16
NS = 16
K = 128
SUB = 25
CH = SUB * K
NBLK = N_EDGES // (NS * CH)
BLK_PER_TILE = N_EDGES // (NS * K)
ROWS_PER_TILE = N_NODES // NS
ZCH = 125


def _edge_kernel_call(src2d, dst2d, ea_t, tables, coef):
    mesh = plsc.VectorSubcoreMesh(core_axis_name="c", subcore_axis_name="s")

    @functools.partial(
        pl.kernel,
        mesh=mesh,
        compiler_params=pltpu.CompilerParams(use_tc_tiling_on_sc=False),
        out_type=jax.ShapeDtypeStruct((2 * N_NODES, F), jnp.float32),
        scratch_types=[
            pltpu.VMEM((SUB, K), jnp.int32),
            pltpu.VMEM((SUB, K), jnp.int32),
            [pltpu.VMEM((CH,), jnp.float32)] * 3,
            [pltpu.VMEM((K, F), jnp.float32)] * 3,
            [pltpu.VMEM((K, F), jnp.float32)] * 3,
            pltpu.VMEM((ZCH, F), jnp.float32),
            pltpu.VMEM((3, F), jnp.float32),
            pltpu.VMEM_SHARED((N_NODES, F), jnp.float32),
            [pltpu.SemaphoreType.DMA] * 3,
            [pltpu.SemaphoreType.DMA] * 3,
        ],
    )
    def k(src_hbm, dst_hbm, ea_hbm, tab_hbm, coef_hbm, out_hbm,
          src_v, dst_v, ea_v, rows_v, z_v, zbuf_v, coef_v, acc_sh, sem_g, sem_s):
        c = lax.axis_index("c")
        s = lax.axis_index("s")

        @plsc.parallel_loop(0, ZCH, 1, unroll=4)
        def zrow(i):
            zbuf_v[i, :] = jnp.zeros((F,), jnp.float32)

        def zcopy(t, _):
            pltpu.sync_copy(zbuf_v, acc_sh.at[pl.ds(s * ROWS_PER_TILE + t * ZCH, ZCH)])
            return _
        lax.fori_loop(0, ROWS_PER_TILE // ZCH, zcopy, 0)

        pltpu.sync_copy(coef_hbm.at[c], coef_v)
        plsc.subcore_barrier()

        c0 = coef_v[0, :]
        c1 = coef_v[1, :]
        c2 = coef_v[2, :]
        row_off = c * N_NODES

        def issue_gather(j, b):
            pltpu.async_copy(tab_hbm.at[src_v.at[j]], rows_v[b], sem_g[b])

        def wait_gather(j, b):
            pltpu.make_async_copy(tab_hbm.at[src_v.at[j]], rows_v[b], sem_g[b]).wait()

        def issue_scatter(j, b):
            pltpu.async_copy(z_v[b], acc_sh.at[dst_v.at[j]], sem_s[b], add=True)

        def wait_scatter(j, b):
            pltpu.make_async_copy(z_v[b], acc_sh.at[dst_v.at[j]], sem_s[b]).wait()

        def compute(j, b):
            @plsc.parallel_loop(0, K // 16, 1, unroll=2)
            def grp(g):
                off = j * K + 16 * g
                ve0 = ea_v[0][pl.ds(off, 16)]
                ve1 = ea_v[1][pl.ds(off, 16)]
                ve2 = ea_v[2][pl.ds(off, 16)]
                for u in range(16):
                    e = 16 * g + u
                    gv = rows_v[b][e, :]
                    z_v[b][e, :] = jnp.maximum(
                        gv + ve0[u] * c0 + ve1[u] * c1 + ve2[u] * c2, 0.0)

        def block(i, _):
            blk0 = s * BLK_PER_TILE + i * SUB
            pltpu.sync_copy(src_hbm.at[pl.ds(blk0, SUB)], src_v)
            pltpu.sync_copy(dst_hbm.at[pl.ds(blk0, SUB)], dst_v)
            for t in range(3):
                pltpu.sync_copy(ea_hbm.at[t, pl.ds(blk0 * K, CH)], ea_v[t])

            @plsc.parallel_loop(0, SUB * (K // 16), 1, unroll=4)
            def shift(t):
                r = t // (K // 16)
                l = (t % (K // 16)) * 16
                src_v[r, pl.ds(l, 16)] = src_v[r, pl.ds(l, 16)] + row_off

            issue_gather(0, 0)
            issue_gather(1, 1)

            def triple(jj, _):
                for b in range(3):
                    j = 3 * jj + b
                    wait_gather(j, b)
                    if b < 2:
                        issue_gather(j + 2, (b + 2) % 3)
                    else:
                        @pl.when(jj < (SUB - 1) // 3 - 1)
                        def _g():
                            issue_gather(j + 2, (b + 2) % 3)

                    @pl.when(jj > 0)
                    def _w():
                        wait_scatter(j - 3, b)
                    compute(j, b)
                    issue_scatter(j, b)
                return _
            lax.fori_loop(0, (SUB - 1) // 3, triple, 0)

            wait_gather(SUB - 1, 0)
            wait_scatter(SUB - 4, 0)
            compute(SUB - 1, 0)
            issue_scatter(SUB - 1, 0)
            wait_scatter(SUB - 1, 0)
            wait_scatter(SUB - 3, 1)
            wait_scatter(SUB - 2, 2)
            return _
        lax.fori_loop(0, NBLK, block, 0)

        plsc.subcore_barrier()
        r0 = s * ROWS_PER_TILE
        pltpu.sync_copy(acc_sh.at[pl.ds(r0, ROWS_PER_TILE)],
                        out_hbm.at[pl.ds(row_off + r0, ROWS_PER_TILE)])

    return k(src2d, dst2d, ea_t, tables, coef)


BN = 10000


def _hi_chunk(pre):
    n = pre.shape[0]
    return jnp.concatenate(
        [pre[:, 16:20],
         jnp.ones((n, 1), jnp.float32),
         jnp.zeros((n, 11), jnp.float32)], axis=1)


def _prep1_body(x_ref, w_ref, b_ref, o_ref):
    xb = x_ref[...]
    w = w_ref[...]
    pre = lax.dot_general(xb, w[:, :4], (((1,), (1,)), ((), ())),
                          preferred_element_type=jnp.float32) + b_ref[...][None, :]
    o_ref[0] = pre[:, :16]
    o_ref[1] = _hi_chunk(pre)


def _mean_head(acc_ref, w2_ref, b2_ref):
    lo = acc_ref[0]
    hi = acc_ref[1]
    cnt = hi[:, 4]
    ssum = jnp.concatenate([lo, hi[:, :4]], axis=1)
    mean = ssum / jnp.maximum(cnt, 1.0)[:, None]
    h = lax.dot_general(mean, w2_ref[...], (((1,), (1,)), ((), ())),
                        preferred_element_type=jnp.float32)
    return h + b2_ref[...][None, :] * jnp.minimum(cnt, 1.0)[:, None]


def _mid_body(acc_ref, w2_ref, b2_ref, w1n_ref, b1n_ref, o_ref):
    h = jnp.maximum(_mean_head(acc_ref, w2_ref, b2_ref), 0.0)
    w1n = w1n_ref[...]
    pre = lax.dot_general(h, w1n[:, :20], (((1,), (1,)), ((), ())),
                          preferred_element_type=jnp.float32) + b1n_ref[...][None, :]
    o_ref[0] = pre[:, :16]
    o_ref[1] = _hi_chunk(pre)


def _final_body(acc_ref, w2_ref, b2_ref, cw_ref, cb_ref, o_ref):
    h = _mean_head(acc_ref, w2_ref, b2_ref)
    logits = lax.dot_general(h, cw_ref[...], (((1,), (1,)), ((), ())),
                             preferred_element_type=jnp.float32) + cb_ref[...][None, :]
    o_ref[...] = jax.nn.sigmoid(logits)


def _full(shape):
    return pl.BlockSpec(shape, lambda i: tuple(0 for _ in shape))


def _node_call(body, ins, in_specs, out_shape, out_spec):
    return pl.pallas_call(
        body,
        grid=(N_NODES // BN,),
        in_specs=in_specs,
        out_specs=out_spec,
        out_shape=out_shape,
    )(*ins)


def kernel(x, edge_index, edge_attr, p1_W1, p1_b1, p1_W2, p1_b2,
           p2_W1, p2_b1, p2_W2, p2_b2, cls_W, cls_b):
    src = edge_index[0]
    dst = edge_index[1]
    src2d = src.reshape(N_EDGES // K, K)
    dst2d = dst.reshape(N_EDGES // K, K)
    ea_t = edge_attr.T

    def coef_of(W1, dh):
        C = W1[:, dh:]
        lo = C[:16].T
        hi = jnp.pad(C[16:20].T, ((0, 0), (0, 12)))
        return jnp.stack([lo, hi])

    tab_spec = pl.BlockSpec((2, BN, F), lambda i: (0, i, 0))
    acc_spec = pl.BlockSpec((2, BN, F), lambda i: (0, i, 0))

    tables1 = _node_call(
        _prep1_body,
        (x, p1_W1, p1_b1),
        [pl.BlockSpec((BN, 4), lambda i: (i, 0)), _full(p1_W1.shape), _full(p1_b1.shape)],
        jax.ShapeDtypeStruct((2, N_NODES, F), jnp.float32),
        tab_spec,
    )
    acc1 = _edge_kernel_call(src2d, dst2d, ea_t,
                             tables1.reshape(2 * N_NODES, F), coef_of(p1_W1, 4))
    acc1 = acc1.reshape(2, N_NODES, F)

    tables2 = _node_call(
        _mid_body,
        (acc1, p1_W2, p1_b2, p2_W1, p2_b1),
        [acc_spec, _full(p1_W2.shape), _full(p1_b2.shape),
         _full(p2_W1.shape), _full(p2_b1.shape)],
        jax.ShapeDtypeStruct((2, N_NODES, F), jnp.float32),
        tab_spec,
    )
    acc2 = _edge_kernel_call(src2d, dst2d, ea_t,
                             tables2.reshape(2 * N_NODES, F), coef_of(p2_W1, 20))
    acc2 = acc2.reshape(2, N_NODES, F)

    return _node_call(
        _final_body,
        (acc2, p2_W2, p2_b2, cls_W, cls_b),
        [acc_spec, _full(p2_W2.shape), _full(p2_b2.shape),
         _full(cls_W.shape), _full(cls_b.shape)],
        jax.ShapeDtypeStruct((N_NODES, 3), jnp.float32),
        pl.BlockSpec((BN, 3), lambda i: (i, 0)),
    )


# speedup vs baseline: 17.0414x; 1.0104x over previous
import functools

import jax
import jax.numpy as jnp
from jax import lax
from jax.experimental import pallas as pl
from jax.experimental.pallas import tpu as pltpu
from jax.experimental.pallas import tpu_sc as plsc

N_NODES = 100000
N_EDGES = 6400000
F = 16
NS = 16
K = 128
SUB = 25
CH = SUB * K
NBLK = N_EDGES // (NS * CH)
BLK_PER_TILE = N_EDGES // (NS * K)
ROWS_PER_TILE = N_NODES // NS
ZCH = 125


def _edge_kernel_call(src2d, dst2d, ea_t, tables, coef):
    mesh = plsc.VectorSubcoreMesh(core_axis_name="c", subcore_axis_name="s")

    @functools.partial(
        pl.kernel,
        mesh=mesh,
        compiler_params=pltpu.CompilerParams(use_tc_tiling_on_sc=False),
        out_type=jax.ShapeDtypeStruct((2 * N_NODES, F), jnp.float32),
        scratch_types=[
            pltpu.VMEM((SUB, K), jnp.int32),
            pltpu.VMEM((SUB, K), jnp.int32),
            [pltpu.VMEM((CH,), jnp.float32)] * 3,
            [pltpu.VMEM((K, F), jnp.float32)] * 3,
            [pltpu.VMEM((K, F), jnp.float32)] * 3,
            pltpu.VMEM((ZCH, F), jnp.float32),
            pltpu.VMEM((3, F), jnp.float32),
            pltpu.VMEM_SHARED((N_NODES, F), jnp.float32),
            [pltpu.SemaphoreType.DMA] * 3,
            [pltpu.SemaphoreType.DMA] * 3,
        ],
    )
    def k(src_hbm, dst_hbm, ea_hbm, tab_hbm, coef_hbm, out_hbm,
          src_v, dst_v, ea_v, rows_v, z_v, zbuf_v, coef_v, acc_sh, sem_g, sem_s):
        c = lax.axis_index("c")
        s = lax.axis_index("s")

        @plsc.parallel_loop(0, ZCH, 1, unroll=4)
        def zrow(i):
            zbuf_v[i, :] = jnp.zeros((F,), jnp.float32)

        def zcopy(t, _):
            pltpu.sync_copy(zbuf_v, acc_sh.at[pl.ds(s * ROWS_PER_TILE + t * ZCH, ZCH)])
            return _
        lax.fori_loop(0, ROWS_PER_TILE // ZCH, zcopy, 0)

        pltpu.sync_copy(coef_hbm.at[c], coef_v)
        plsc.subcore_barrier()

        c0 = coef_v[0, :]
        c1 = coef_v[1, :]
        c2 = coef_v[2, :]
        row_off = c * N_NODES

        def issue_gather(j, b):
            pltpu.async_copy(tab_hbm.at[src_v.at[j]], rows_v[b], sem_g[b])

        def wait_gather(j, b):
            pltpu.make_async_copy(tab_hbm.at[src_v.at[j]], rows_v[b], sem_g[b]).wait()

        def issue_scatter(j, b):
            pltpu.async_copy(z_v[b], acc_sh.at[dst_v.at[j]], sem_s[b], add=True)

        def wait_scatter(j, b):
            pltpu.make_async_copy(z_v[b], acc_sh.at[dst_v.at[j]], sem_s[b]).wait()

        def compute(j, b):
            @plsc.parallel_loop(0, K // 16, 1, unroll=4)
            def grp(g):
                off = j * K + 16 * g
                ve0 = ea_v[0][pl.ds(off, 16)]
                ve1 = ea_v[1][pl.ds(off, 16)]
                ve2 = ea_v[2][pl.ds(off, 16)]
                for u in range(16):
                    e = 16 * g + u
                    gv = rows_v[b][e, :]
                    z_v[b][e, :] = jnp.maximum(
                        gv + ve0[u] * c0 + ve1[u] * c1 + ve2[u] * c2, 0.0)

        def block(i, _):
            blk0 = s * BLK_PER_TILE + i * SUB
            pltpu.sync_copy(src_hbm.at[pl.ds(blk0, SUB)], src_v)
            pltpu.sync_copy(dst_hbm.at[pl.ds(blk0, SUB)], dst_v)
            for t in range(3):
                pltpu.sync_copy(ea_hbm.at[t, pl.ds(blk0 * K, CH)], ea_v[t])

            @plsc.parallel_loop(0, SUB * (K // 16), 1, unroll=4)
            def shift(t):
                r = t // (K // 16)
                l = (t % (K // 16)) * 16
                src_v[r, pl.ds(l, 16)] = src_v[r, pl.ds(l, 16)] + row_off

            issue_gather(0, 0)
            issue_gather(1, 1)

            def triple(jj, _):
                for b in range(3):
                    j = 3 * jj + b
                    wait_gather(j, b)
                    if b < 2:
                        issue_gather(j + 2, (b + 2) % 3)
                    else:
                        @pl.when(jj < (SUB - 1) // 3 - 1)
                        def _g():
                            issue_gather(j + 2, (b + 2) % 3)

                    @pl.when(jj > 0)
                    def _w():
                        wait_scatter(j - 3, b)
                    compute(j, b)
                    issue_scatter(j, b)
                return _
            lax.fori_loop(0, (SUB - 1) // 3, triple, 0)

            wait_gather(SUB - 1, 0)
            wait_scatter(SUB - 4, 0)
            compute(SUB - 1, 0)
            issue_scatter(SUB - 1, 0)
            wait_scatter(SUB - 1, 0)
            wait_scatter(SUB - 3, 1)
            wait_scatter(SUB - 2, 2)
            return _
        lax.fori_loop(0, NBLK, block, 0)

        plsc.subcore_barrier()
        r0 = s * ROWS_PER_TILE
        pltpu.sync_copy(acc_sh.at[pl.ds(r0, ROWS_PER_TILE)],
                        out_hbm.at[pl.ds(row_off + r0, ROWS_PER_TILE)])

    return k(src2d, dst2d, ea_t, tables, coef)


BN = 10000


def _hi_chunk(pre):
    n = pre.shape[0]
    return jnp.concatenate(
        [pre[:, 16:20],
         jnp.ones((n, 1), jnp.float32),
         jnp.zeros((n, 11), jnp.float32)], axis=1)


def _prep1_body(x_ref, w_ref, b_ref, o_ref):
    xb = x_ref[...]
    w = w_ref[...]
    pre = lax.dot_general(xb, w[:, :4], (((1,), (1,)), ((), ())),
                          preferred_element_type=jnp.float32) + b_ref[...][None, :]
    o_ref[0] = pre[:, :16]
    o_ref[1] = _hi_chunk(pre)


def _mean_head(acc_ref, w2_ref, b2_ref):
    lo = acc_ref[0]
    hi = acc_ref[1]
    cnt = hi[:, 4]
    ssum = jnp.concatenate([lo, hi[:, :4]], axis=1)
    mean = ssum / jnp.maximum(cnt, 1.0)[:, None]
    h = lax.dot_general(mean, w2_ref[...], (((1,), (1,)), ((), ())),
                        preferred_element_type=jnp.float32)
    return h + b2_ref[...][None, :] * jnp.minimum(cnt, 1.0)[:, None]


def _mid_body(acc_ref, w2_ref, b2_ref, w1n_ref, b1n_ref, o_ref):
    h = jnp.maximum(_mean_head(acc_ref, w2_ref, b2_ref), 0.0)
    w1n = w1n_ref[...]
    pre = lax.dot_general(h, w1n[:, :20], (((1,), (1,)), ((), ())),
                          preferred_element_type=jnp.float32) + b1n_ref[...][None, :]
    o_ref[0] = pre[:, :16]
    o_ref[1] = _hi_chunk(pre)


def _final_body(acc_ref, w2_ref, b2_ref, cw_ref, cb_ref, o_ref):
    h = _mean_head(acc_ref, w2_ref, b2_ref)
    logits = lax.dot_general(h, cw_ref[...], (((1,), (1,)), ((), ())),
                             preferred_element_type=jnp.float32) + cb_ref[...][None, :]
    o_ref[...] = jax.nn.sigmoid(logits)


def _full(shape):
    return pl.BlockSpec(shape, lambda i: tuple(0 for _ in shape))


def _node_call(body, ins, in_specs, out_shape, out_spec):
    return pl.pallas_call(
        body,
        grid=(N_NODES // BN,),
        in_specs=in_specs,
        out_specs=out_spec,
        out_shape=out_shape,
    )(*ins)


def kernel(x, edge_index, edge_attr, p1_W1, p1_b1, p1_W2, p1_b2,
           p2_W1, p2_b1, p2_W2, p2_b2, cls_W, cls_b):
    src = edge_index[0]
    dst = edge_index[1]
    src2d = src.reshape(N_EDGES // K, K)
    dst2d = dst.reshape(N_EDGES // K, K)
    ea_t = edge_attr.T

    def coef_of(W1, dh):
        C = W1[:, dh:]
        lo = C[:16].T
        hi = jnp.pad(C[16:20].T, ((0, 0), (0, 12)))
        return jnp.stack([lo, hi])

    tab_spec = pl.BlockSpec((2, BN, F), lambda i: (0, i, 0))
    acc_spec = pl.BlockSpec((2, BN, F), lambda i: (0, i, 0))

    tables1 = _node_call(
        _prep1_body,
        (x, p1_W1, p1_b1),
        [pl.BlockSpec((BN, 4), lambda i: (i, 0)), _full(p1_W1.shape), _full(p1_b1.shape)],
        jax.ShapeDtypeStruct((2, N_NODES, F), jnp.float32),
        tab_spec,
    )
    acc1 = _edge_kernel_call(src2d, dst2d, ea_t,
                             tables1.reshape(2 * N_NODES, F), coef_of(p1_W1, 4))
    acc1 = acc1.reshape(2, N_NODES, F)

    tables2 = _node_call(
        _mid_body,
        (acc1, p1_W2, p1_b2, p2_W1, p2_b1),
        [acc_spec, _full(p1_W2.shape), _full(p1_b2.shape),
         _full(p2_W1.shape), _full(p2_b1.shape)],
        jax.ShapeDtypeStruct((2, N_NODES, F), jnp.float32),
        tab_spec,
    )
    acc2 = _edge_kernel_call(src2d, dst2d, ea_t,
                             tables2.reshape(2 * N_NODES, F), coef_of(p2_W1, 20))
    acc2 = acc2.reshape(2, N_NODES, F)

    return _node_call(
        _final_body,
        (acc2, p2_W2, p2_b2, cls_W, cls_b),
        [acc_spec, _full(p2_W2.shape), _full(p2_b2.shape),
         _full(cls_W.shape), _full(cls_b.shape)],
        jax.ShapeDtypeStruct((N_NODES, 3), jnp.float32),
        pl.BlockSpec((BN, 3), lambda i: (i, 0)),
    )


# speedup vs baseline: 23.6996x; 1.3907x over previous
import functools

import jax
import jax.numpy as jnp
from jax import lax
from jax.experimental import pallas as pl
from jax.experimental.pallas import tpu as pltpu
from jax.experimental.pallas import tpu_sc as plsc

N_NODES = 100000
N_EDGES = 6400000
F = 16
NS = 16
K = 128
SUB = 25
CH = SUB * K
NBLK = N_EDGES // (NS * CH)
BLK_PER_TILE = N_EDGES // (NS * K)
ROWS_PER_TILE = N_NODES // NS
ZCH = 125


def _edge_kernel_call(src2d, dst2d, ea_t, tables, coef):
    mesh = plsc.VectorSubcoreMesh(core_axis_name="c", subcore_axis_name="s")

    @functools.partial(
        pl.kernel,
        mesh=mesh,
        compiler_params=pltpu.CompilerParams(use_tc_tiling_on_sc=False),
        out_type=jax.ShapeDtypeStruct((2 * N_NODES, F), jnp.float32),
        scratch_types=[
            pltpu.VMEM((SUB, K), jnp.int32),
            pltpu.VMEM((SUB, K), jnp.int32),
            pltpu.VMEM((SUB, 3, K), jnp.float32),
            [pltpu.VMEM((K, F), jnp.float32)] * 3,
            [pltpu.VMEM((K, F), jnp.float32)] * 3,
            pltpu.VMEM((ZCH, F), jnp.float32),
            pltpu.VMEM((3, F), jnp.float32),
            pltpu.VMEM_SHARED((N_NODES, F), jnp.float32),
            [pltpu.SemaphoreType.DMA] * 3,
            [pltpu.SemaphoreType.DMA] * 3,
        ],
    )
    def k(src_hbm, dst_hbm, ea_hbm, tab_hbm, coef_hbm, out_hbm,
          src_v, dst_v, ea_v, rows_v, z_v, zbuf_v, coef_v, acc_sh, sem_g, sem_s):
        c = lax.axis_index("c")
        s = lax.axis_index("s")

        @plsc.parallel_loop(0, ZCH, 1, unroll=4)
        def zrow(i):
            zbuf_v[i, :] = jnp.zeros((F,), jnp.float32)

        def zcopy(t, _):
            pltpu.sync_copy(zbuf_v, acc_sh.at[pl.ds(s * ROWS_PER_TILE + t * ZCH, ZCH)])
            return _
        lax.fori_loop(0, ROWS_PER_TILE // ZCH, zcopy, 0)

        pltpu.sync_copy(coef_hbm.at[c], coef_v)
        plsc.subcore_barrier()

        c0 = coef_v[0, :]
        c1 = coef_v[1, :]
        c2 = coef_v[2, :]
        row_off = c * N_NODES

        def issue_gather(j, b):
            pltpu.async_copy(tab_hbm.at[src_v.at[j]], rows_v[b], sem_g[b])

        def wait_gather(j, b):
            pltpu.make_async_copy(tab_hbm.at[src_v.at[j]], rows_v[b], sem_g[b]).wait()

        def issue_scatter(j, b):
            pltpu.async_copy(z_v[b], acc_sh.at[dst_v.at[j]], sem_s[b], add=True)

        def wait_scatter(j, b):
            pltpu.make_async_copy(z_v[b], acc_sh.at[dst_v.at[j]], sem_s[b]).wait()

        def compute(j, b):
            @plsc.parallel_loop(0, K // 16, 1, unroll=4)
            def grp(g):
                off = 16 * g
                ve0 = ea_v[j, 0, pl.ds(off, 16)]
                ve1 = ea_v[j, 1, pl.ds(off, 16)]
                ve2 = ea_v[j, 2, pl.ds(off, 16)]
                for u in range(16):
                    e = 16 * g + u
                    gv = rows_v[b][e, :]
                    z_v[b][e, :] = jnp.maximum(
                        gv + ve0[u] * c0 + ve1[u] * c1 + ve2[u] * c2, 0.0)

        def block(i, _):
            blk0 = s * BLK_PER_TILE + i * SUB
            pltpu.sync_copy(src_hbm.at[pl.ds(blk0, SUB)], src_v)
            pltpu.sync_copy(dst_hbm.at[pl.ds(blk0, SUB)], dst_v)
            pltpu.sync_copy(ea_hbm.at[pl.ds(blk0, SUB)], ea_v)

            @plsc.parallel_loop(0, SUB * (K // 16), 1, unroll=4)
            def shift(t):
                r = t // (K // 16)
                l = (t % (K // 16)) * 16
                src_v[r, pl.ds(l, 16)] = src_v[r, pl.ds(l, 16)] + row_off

            issue_gather(0, 0)
            issue_gather(1, 1)

            def triple(jj, _):
                for b in range(3):
                    j = 3 * jj + b
                    wait_gather(j, b)
                    if b < 2:
                        issue_gather(j + 2, (b + 2) % 3)
                    else:
                        @pl.when(jj < (SUB - 1) // 3 - 1)
                        def _g():
                            issue_gather(j + 2, (b + 2) % 3)

                    @pl.when(jj > 0)
                    def _w():
                        wait_scatter(j - 3, b)
                    compute(j, b)
                    issue_scatter(j, b)
                return _
            lax.fori_loop(0, (SUB - 1) // 3, triple, 0)

            wait_gather(SUB - 1, 0)
            wait_scatter(SUB - 4, 0)
            compute(SUB - 1, 0)
            issue_scatter(SUB - 1, 0)
            wait_scatter(SUB - 1, 0)
            wait_scatter(SUB - 3, 1)
            wait_scatter(SUB - 2, 2)
            return _
        lax.fori_loop(0, NBLK, block, 0)

        plsc.subcore_barrier()
        r0 = s * ROWS_PER_TILE
        pltpu.sync_copy(acc_sh.at[pl.ds(r0, ROWS_PER_TILE)],
                        out_hbm.at[pl.ds(row_off + r0, ROWS_PER_TILE)])

    return k(src2d, dst2d, ea_t, tables, coef)


BN = 10000


def _hi_chunk(pre):
    n = pre.shape[0]
    return jnp.concatenate(
        [pre[:, 16:20],
         jnp.ones((n, 1), jnp.float32),
         jnp.zeros((n, 11), jnp.float32)], axis=1)


def _prep1_body(x_ref, w_ref, b_ref, o_ref):
    xb = x_ref[...]
    w = w_ref[...]
    pre = lax.dot_general(xb, w[:, :4], (((1,), (1,)), ((), ())),
                          preferred_element_type=jnp.float32) + b_ref[...][None, :]
    o_ref[0] = pre[:, :16]
    o_ref[1] = _hi_chunk(pre)


def _mean_head(acc_ref, w2_ref, b2_ref):
    lo = acc_ref[0]
    hi = acc_ref[1]
    cnt = hi[:, 4]
    ssum = jnp.concatenate([lo, hi[:, :4]], axis=1)
    mean = ssum / jnp.maximum(cnt, 1.0)[:, None]
    h = lax.dot_general(mean, w2_ref[...], (((1,), (1,)), ((), ())),
                        preferred_element_type=jnp.float32)
    return h + b2_ref[...][None, :] * jnp.minimum(cnt, 1.0)[:, None]


def _mid_body(acc_ref, w2_ref, b2_ref, w1n_ref, b1n_ref, o_ref):
    h = jnp.maximum(_mean_head(acc_ref, w2_ref, b2_ref), 0.0)
    w1n = w1n_ref[...]
    pre = lax.dot_general(h, w1n[:, :20], (((1,), (1,)), ((), ())),
                          preferred_element_type=jnp.float32) + b1n_ref[...][None, :]
    o_ref[0] = pre[:, :16]
    o_ref[1] = _hi_chunk(pre)


def _final_body(acc_ref, w2_ref, b2_ref, cw_ref, cb_ref, o_ref):
    h = _mean_head(acc_ref, w2_ref, b2_ref)
    logits = lax.dot_general(h, cw_ref[...], (((1,), (1,)), ((), ())),
                             preferred_element_type=jnp.float32) + cb_ref[...][None, :]
    o_ref[...] = jax.nn.sigmoid(logits)


def _full(shape):
    return pl.BlockSpec(shape, lambda i: tuple(0 for _ in shape))


def _node_call(body, ins, in_specs, out_shape, out_spec):
    return pl.pallas_call(
        body,
        grid=(N_NODES // BN,),
        in_specs=in_specs,
        out_specs=out_spec,
        out_shape=out_shape,
    )(*ins)


def kernel(x, edge_index, edge_attr, p1_W1, p1_b1, p1_W2, p1_b2,
           p2_W1, p2_b1, p2_W2, p2_b2, cls_W, cls_b):
    src = edge_index[0]
    dst = edge_index[1]
    src2d = src.reshape(N_EDGES // K, K)
    dst2d = dst.reshape(N_EDGES // K, K)
    ea_t = jnp.swapaxes(edge_attr.reshape(N_EDGES // K, K, 3), 1, 2)

    def coef_of(W1, dh):
        C = W1[:, dh:]
        lo = C[:16].T
        hi = jnp.pad(C[16:20].T, ((0, 0), (0, 12)))
        return jnp.stack([lo, hi])

    tab_spec = pl.BlockSpec((2, BN, F), lambda i: (0, i, 0))
    acc_spec = pl.BlockSpec((2, BN, F), lambda i: (0, i, 0))

    tables1 = _node_call(
        _prep1_body,
        (x, p1_W1, p1_b1),
        [pl.BlockSpec((BN, 4), lambda i: (i, 0)), _full(p1_W1.shape), _full(p1_b1.shape)],
        jax.ShapeDtypeStruct((2, N_NODES, F), jnp.float32),
        tab_spec,
    )
    acc1 = _edge_kernel_call(src2d, dst2d, ea_t,
                             tables1.reshape(2 * N_NODES, F), coef_of(p1_W1, 4))
    acc1 = acc1.reshape(2, N_NODES, F)

    tables2 = _node_call(
        _mid_body,
        (acc1, p1_W2, p1_b2, p2_W1, p2_b1),
        [acc_spec, _full(p1_W2.shape), _full(p1_b2.shape),
         _full(p2_W1.shape), _full(p2_b1.shape)],
        jax.ShapeDtypeStruct((2, N_NODES, F), jnp.float32),
        tab_spec,
    )
    acc2 = _edge_kernel_call(src2d, dst2d, ea_t,
                             tables2.reshape(2 * N_NODES, F), coef_of(p2_W1, 20))
    acc2 = acc2.reshape(2, N_NODES, F)

    return _node_call(
        _final_body,
        (acc2, p2_W2, p2_b2, cls_W, cls_b),
        [acc_spec, _full(p2_W2.shape), _full(p2_b2.shape),
         _full(cls_W.shape), _full(cls_b.shape)],
        jax.ShapeDtypeStruct((N_NODES, 3), jnp.float32),
        pl.BlockSpec((BN, 3), lambda i: (i, 0)),
    )
